# baseline (speedup 1.0000x reference)
import functools

import jax
import jax.numpy as jnp
from jax import lax
from jax.experimental import pallas as pl
from jax.experimental.pallas import tpu as pltpu
from jax.experimental.pallas import tpu_sc as plsc

_D = 128
_CHUNK_B = 16


def _make_sc_lookup(batch: int, seq: int, n_items: int):
    info = plsc.get_sparse_core_info()
    nw = info.num_cores * info.num_subcores
    b_per_w = batch // nw
    rows_per_w = b_per_w * seq
    n_chunks = b_per_w // _CHUNK_B
    chunk_rows = _CHUNK_B * seq
    assert batch % nw == 0 and b_per_w % (2 * _CHUNK_B) == 0
    mesh = plsc.VectorSubcoreMesh(core_axis_name="c", subcore_axis_name="s")

    @functools.partial(
        pl.kernel,
        mesh=mesh,
        out_type=jax.ShapeDtypeStruct((batch, seq, _D), jnp.float32),
        scratch_types=[
            pltpu.VMEM_SHARED((n_items, _D), jnp.float32),
            pltpu.VMEM((_CHUNK_B, seq), jnp.int32),
            pltpu.VMEM((_CHUNK_B, seq), jnp.int32),
            pltpu.VMEM((_CHUNK_B, seq, _D), jnp.float32),
            pltpu.VMEM((_CHUNK_B, seq, _D), jnp.float32),
            pltpu.SemaphoreType.DMA,
            pltpu.SemaphoreType.DMA,
            pltpu.SemaphoreType.DMA,
            pltpu.SemaphoreType.DMA,
            pltpu.SemaphoreType.DMA,
            pltpu.SemaphoreType.DMA,
        ],
    )
    def lookup(idx_hbm, table_hbm, out_hbm, table_sh, ibuf0, ibuf1,
               buf0, buf1, isem0, isem1, gsem0, gsem1, osem0, osem1):
        cid = lax.axis_index("c")
        sid = lax.axis_index("s")
        wid = sid * info.num_cores + cid
        ibufs = (ibuf0, ibuf1)
        bufs = (buf0, buf1)
        isems = (isem0, isem1)
        gsems = (gsem0, gsem1)
        osems = (osem0, osem1)

        @pl.when(sid == 0)
        def _stage_table():
            pltpu.sync_copy(table_hbm, table_sh)

        plsc.subcore_barrier()

        idx_descs = {}
        gather_descs = {}
        scatter_descs = {}

        def fire_idx(c):
            b = c % 2
            idx_descs[c] = pltpu.async_copy(
                idx_hbm.at[wid, pl.ds(c * _CHUNK_B, _CHUNK_B)],
                ibufs[b], isems[b])

        def fire_gathers(c):
            b = c % 2
            ds = []
            for k in range(_CHUNK_B):
                ds.append(pltpu.async_copy(
                    table_sh.at[ibufs[b].at[k]],
                    bufs[b].at[k],
                    gsems[b]))
            gather_descs[c] = ds

        def fire_scatter(c):
            b = c % 2
            scatter_descs[c] = pltpu.async_copy(
                bufs[b],
                out_hbm.at[pl.ds(wid * b_per_w + c * _CHUNK_B, _CHUNK_B)],
                osems[b])

        fire_idx(0)
        fire_idx(1)
        idx_descs.pop(0).wait()
        fire_gathers(0)
        for c in range(n_chunks):
            for d in gather_descs.pop(c):
                d.wait()
            fire_scatter(c)
            if c + 2 < n_chunks:
                fire_idx(c + 2)
            if c + 1 < n_chunks:
                if c >= 1:
                    scatter_descs.pop(c - 1).wait()
                idx_descs.pop(c + 1).wait()
                fire_gathers(c + 1)
        scatter_descs.pop(n_chunks - 1).wait()

    def run(x, table):
        idx2 = x.reshape(nw, b_per_w, seq)
        return lookup(idx2, table)

    return run


def kernel(x, table):
    b, s = x.shape
    half = b // 2
    lookup_half = _make_sc_lookup(half, s, table.shape[0])
    o1 = lookup_half(x[:half], table)
    o2 = lookup_half(x[half:], table)
    return jnp.concatenate([o1, o2], axis=0)


# speedup vs baseline: 1.6706x; 1.6706x over previous
import functools

import jax
import jax.numpy as jnp
from jax import lax
from jax.experimental import pallas as pl
from jax.experimental.pallas import tpu as pltpu
from jax.experimental.pallas import tpu_sc as plsc

_D = 128
_CHUNK_B = 16


def _make_sc_lookup(batch: int, seq: int, n_items: int):
    info = plsc.get_sparse_core_info()
    nw = info.num_cores * info.num_subcores
    b_per_w = batch // nw
    rows_per_w = b_per_w * seq
    n_chunks = b_per_w // _CHUNK_B
    chunk_rows = _CHUNK_B * seq
    assert batch % nw == 0 and b_per_w % (2 * _CHUNK_B) == 0
    mesh = plsc.VectorSubcoreMesh(core_axis_name="c", subcore_axis_name="s")

    @functools.partial(
        pl.kernel,
        mesh=mesh,
        out_type=jax.ShapeDtypeStruct((batch, seq, _D), jnp.float32),
        scratch_types=[
            pltpu.VMEM_SHARED((n_items, _D), jnp.float32),
            pltpu.VMEM((_CHUNK_B, seq), jnp.int32),
            pltpu.VMEM((_CHUNK_B, seq), jnp.int32),
            pltpu.VMEM((_CHUNK_B, seq, _D), jnp.float32),
            pltpu.VMEM((_CHUNK_B, seq, _D), jnp.float32),
            pltpu.SemaphoreType.DMA,
            pltpu.SemaphoreType.DMA,
            pltpu.SemaphoreType.DMA,
            pltpu.SemaphoreType.DMA,
            pltpu.SemaphoreType.DMA,
            pltpu.SemaphoreType.DMA,
        ],
    )
    def lookup(idx_hbm, table_hbm, out_hbm, table_sh, ibuf0, ibuf1,
               buf0, buf1, isem0, isem1, gsem0, gsem1, osem0, osem1):
        cid = lax.axis_index("c")
        sid = lax.axis_index("s")
        wid = sid * info.num_cores + cid
        ibufs = (ibuf0, ibuf1)
        bufs = (buf0, buf1)
        isems = (isem0, isem1)
        gsems = (gsem0, gsem1)
        osems = (osem0, osem1)

        @pl.when(sid == 0)
        def _stage_table():
            pltpu.sync_copy(table_hbm, table_sh)

        plsc.subcore_barrier()

        idx_descs = {}
        gather_descs = {}
        scatter_descs = {}

        def fire_idx(c):
            b = c % 2
            idx_descs[c] = pltpu.async_copy(
                idx_hbm.at[wid, pl.ds(c * _CHUNK_B, _CHUNK_B)],
                ibufs[b], isems[b])

        def fire_gathers(c):
            b = c % 2
            ds = []
            for k in range(_CHUNK_B):
                ds.append(pltpu.async_copy(
                    table_sh.at[ibufs[b].at[k]],
                    bufs[b].at[k],
                    gsems[b]))
            gather_descs[c] = ds

        def fire_scatter(c):
            b = c % 2
            scatter_descs[c] = pltpu.async_copy(
                bufs[b],
                out_hbm.at[pl.ds(wid * b_per_w + c * _CHUNK_B, _CHUNK_B)],
                osems[b])

        fire_idx(0)
        fire_idx(1)
        idx_descs.pop(0).wait()
        fire_gathers(0)
        for c in range(n_chunks):
            for d in gather_descs.pop(c):
                d.wait()
            fire_scatter(c)
            if c + 2 < n_chunks:
                fire_idx(c + 2)
            if c + 1 < n_chunks:
                if c >= 1:
                    scatter_descs.pop(c - 1).wait()
                idx_descs.pop(c + 1).wait()
                fire_gathers(c + 1)
        scatter_descs.pop(n_chunks - 1).wait()

    def run(x, table):
        idx2 = x.reshape(nw, b_per_w, seq)
        return lookup(idx2, table)

    return run


def kernel(x, table):
    b, s = x.shape
    return _make_sc_lookup(b, s, table.shape[0])(x, table)
